# baseline (speedup 1.0000x reference)
import functools

import jax
import jax.numpy as jnp
from jax import lax
from jax.experimental import pallas as pl
from jax.experimental.pallas import tpu as pltpu
from jax.experimental.pallas import tpu_sc as plsc

B, S, D = 4, 2048, 1024
N = 512
WD = 128
OUT_D = 2 * D + WD
G = B * N
NC, NS, L = 2, 16, 16
NW = NC * NS
SPW = G // NW
CH = 16
NCHUNK = SPW // CH
K = 3

_mesh = plsc.VectorSubcoreMesh(core_axis_name="c", subcore_axis_name="s")


@functools.partial(
    pl.kernel,
    mesh=_mesh,
    out_type=jax.ShapeDtypeStruct((G, OUT_D), jnp.float32),
    scratch_types=(
        [
            pltpu.VMEM((SPW * 2,), jnp.int32),
            pltpu.VMEM((NCHUNK, 2 * CH), jnp.int32),
            pltpu.VMEM((NCHUNK, CH), jnp.int32),
        ]
        + [pltpu.VMEM((2 * CH, D), jnp.float32) for _ in range(K)]
        + [pltpu.VMEM((CH, WD), jnp.float32) for _ in range(K)]
        + [pltpu.SemaphoreType.DMA for _ in range(3 * K)]
    ),
)
def _span_extract(seq_hbm, spans_hbm, wtab_hbm, out_hbm,
                  spans_v, midx_v, widx_v, *bufs):
    seqb = bufs[:K]
    wrow = bufs[K:2 * K]
    gsem = [bufs[2 * K + 2 * k:2 * K + 2 * k + 2] for k in range(K)]
    wsem = bufs[4 * K:5 * K]
    wid = lax.axis_index("s") * NC + lax.axis_index("c")
    base = wid * SPW
    boff = (base // N) * S
    pltpu.sync_copy(spans_hbm.at[pl.ds(base * 2, SPW * 2)], spans_v)
    lane = lax.iota(jnp.int32, L)
    duo = (2 * lane) & (L - 1)
    half = lane < (L // 2)

    def _pick(v, idx):
        return lax.gather(
            v, idx[:, None],
            dimension_numbers=lax.GatherDimensionNumbers(
                offset_dims=(), collapsed_slice_dims=(0,),
                start_index_map=(0,)),
            slice_sizes=(1,),
            mode=lax.GatherScatterMode.PROMISE_IN_BOUNDS)

    for c in range(NCHUNK):
        a = spans_v[pl.ds(c * 2 * L, L)]
        b = spans_v[pl.ds(c * 2 * L + L, L)]
        s = jnp.where(half, _pick(a, duo), _pick(b, duo))
        e = jnp.where(half, _pick(a, duo + 1), _pick(b, duo + 1))
        midx_v[c, pl.ds(0, CH)] = s + boff
        midx_v[c, pl.ds(CH, CH)] = e + boff
        widx_v[c] = e - s

    gh = [None] * NCHUNK
    wh = [[] for _ in range(K)]

    def fire(c):
        k = c % K
        for h in wh[k]:
            h.wait()
        wh[k] = []
        gh[c] = (
            pltpu.async_copy(wtab_hbm.at[widx_v.at[c]], wrow[k], gsem[k][0]),
            pltpu.async_copy(seq_hbm.at[midx_v.at[c]], seqb[k], gsem[k][1]),
        )

    for c in range(min(K, NCHUNK)):
        fire(c)
    for c in range(NCHUNK):
        k = c % K
        rows = pl.ds(base + c * CH, CH)
        gh[c][0].wait()
        wh[k].append(pltpu.async_copy(
            wrow[k], out_hbm.at[rows, pl.ds(2 * D, WD)], wsem[k]))
        gh[c][1].wait()
        wh[k].append(pltpu.async_copy(
            seqb[k].at[pl.ds(0, CH)],
            out_hbm.at[rows, pl.ds(0, D)], wsem[k]))
        wh[k].append(pltpu.async_copy(
            seqb[k].at[pl.ds(CH, CH)],
            out_hbm.at[rows, pl.ds(D, D)], wsem[k]))
        if c + K < NCHUNK:
            fire(c + K)
    for k in range(K):
        for h in wh[k]:
            h.wait()


def kernel(sequence_tensor, span_indices, width_table):
    seq_flat = sequence_tensor.reshape(B * S, D)
    spans_flat = span_indices.astype(jnp.int32).reshape(G * 2)
    out = _span_extract(seq_flat, spans_flat, width_table)
    return out.reshape(B, N, OUT_D)


# speedup vs baseline: 1.0075x; 1.0075x over previous
import functools

import jax
import jax.numpy as jnp
from jax import lax
from jax.experimental import pallas as pl
from jax.experimental.pallas import tpu as pltpu
from jax.experimental.pallas import tpu_sc as plsc

B, S, D = 4, 2048, 1024
N = 512
WD = 128
OUT_D = 2 * D + WD
G = B * N
NC, NS, L = 2, 16, 16
NW = NC * NS
SPW = G // NW
CH = 16
NCHUNK = SPW // CH
K = 3

_mesh = plsc.VectorSubcoreMesh(core_axis_name="c", subcore_axis_name="s")


@functools.partial(
    pl.kernel,
    mesh=_mesh,
    out_type=jax.ShapeDtypeStruct((G, OUT_D), jnp.float32),
    scratch_types=(
        [
            pltpu.VMEM((SPW * 2,), jnp.int32),
            pltpu.VMEM((NCHUNK, CH), jnp.int32),
            pltpu.VMEM((NCHUNK, CH), jnp.int32),
            pltpu.VMEM((NCHUNK, CH), jnp.int32),
        ]
        + [pltpu.VMEM((CH, OUT_D), jnp.float32) for _ in range(K)]
        + [pltpu.SemaphoreType.DMA for _ in range(4 * K)]
    ),
)
def _span_extract(seq_hbm, spans_hbm, wtab_hbm, out_hbm,
                  spans_v, sidx_v, eidx_v, widx_v, *bufs):
    orow = bufs[:K]
    gsem = [bufs[K + 3 * k:K + 3 * k + 3] for k in range(K)]
    wsem = bufs[4 * K:5 * K]
    wid = lax.axis_index("s") * NC + lax.axis_index("c")
    base = wid * SPW
    boff = (base // N) * S
    pltpu.sync_copy(spans_hbm.at[pl.ds(base * 2, SPW * 2)], spans_v)
    lane = lax.iota(jnp.int32, L)
    duo = (2 * lane) & (L - 1)
    half = lane < (L // 2)

    def _pick(v, idx):
        return lax.gather(
            v, idx[:, None],
            dimension_numbers=lax.GatherDimensionNumbers(
                offset_dims=(), collapsed_slice_dims=(0,),
                start_index_map=(0,)),
            slice_sizes=(1,),
            mode=lax.GatherScatterMode.PROMISE_IN_BOUNDS)

    for c in range(NCHUNK):
        a = spans_v[pl.ds(c * 2 * L, L)]
        b = spans_v[pl.ds(c * 2 * L + L, L)]
        s = jnp.where(half, _pick(a, duo), _pick(b, duo))
        e = jnp.where(half, _pick(a, duo + 1), _pick(b, duo + 1))
        sidx_v[c] = s + boff
        eidx_v[c] = e + boff
        widx_v[c] = e - s

    gh = [None] * NCHUNK
    wh = [[] for _ in range(K)]
    cols = ((0, D), (D, D), (2 * D, WD))

    def fire(c):
        k = c % K
        for h in wh[k]:
            h.wait()
        wh[k] = []
        gh[c] = tuple(
            pltpu.async_copy(src, orow[k].at[:, pl.ds(lo, w)], gsem[k][i])
            for i, ((lo, w), src) in enumerate(zip(
                cols,
                (seq_hbm.at[sidx_v.at[c]], seq_hbm.at[eidx_v.at[c]],
                 wtab_hbm.at[widx_v.at[c]]))))

    for c in range(min(K, NCHUNK)):
        fire(c)
    for c in range(NCHUNK):
        k = c % K
        rows = pl.ds(base + c * CH, CH)
        for i, (lo, w) in enumerate(cols):
            gh[c][i].wait()
            wh[k].append(pltpu.async_copy(
                orow[k].at[:, pl.ds(lo, w)],
                out_hbm.at[rows, pl.ds(lo, w)], wsem[k]))
        if c + K < NCHUNK:
            fire(c + K)
    for k in range(K):
        for h in wh[k]:
            h.wait()


def kernel(sequence_tensor, span_indices, width_table):
    seq_flat = sequence_tensor.reshape(B * S, D)
    spans_flat = span_indices.astype(jnp.int32).reshape(G * 2)
    out = _span_extract(seq_flat, spans_flat, width_table)
    return out.reshape(B, N, OUT_D)
